# Initial kernel scaffold; baseline (speedup 1.0000x reference)
import jax
import jax.numpy as jnp
from jax.experimental import pallas as pl


def kernel(inputs, targets):
    raise NotImplementedError("write your pallas kernel here")


# speedup vs baseline: 2.0755x; 2.0755x over previous
import functools

import numpy as np
import jax
import jax.numpy as jnp
from jax.experimental import pallas as pl
from jax.experimental.pallas import tpu as pltpu

_BINS = 64
_OFF = 2.0 * np.pi / _BINS
_C0 = np.float32(-np.pi + _OFF / 2.0)
_C63 = np.float32(-np.pi + _OFF / 2.0 + 63 * _OFF)
_HPI = np.float32(np.pi / 2.0)

_COLS = 1875
_BROWS = 160


def _shift_left(a, k):
    r, _ = a.shape
    return jnp.concatenate([a[:, k:], jnp.zeros((r, k), a.dtype)], axis=1)


def _body(nblk, total_n, xi_ref, xt_ref, out_ref, acc_s, acc_n):
    i = pl.program_id(0)

    @pl.when(i == 0)
    def _():
        acc_s[...] = jnp.zeros_like(acc_s)
        acc_n[...] = jnp.zeros_like(acc_n)

    xi = xi_ref[...]
    xt = xt_ref[...]

    ds = jnp.sin(xi) - jnp.sin(xt)
    dc = jnp.cos(xi) - jnp.cos(xt)
    ds2 = ds * ds
    dc2 = dc * dc

    nsin = ds2 + _shift_left(ds2, 1) + _shift_left(ds2, 2)
    ncos = dc2 + _shift_left(dc2, 1) + _shift_left(dc2, 2)
    r = jnp.sqrt(nsin) + jnp.sqrt(ncos)

    lane = jax.lax.broadcasted_iota(jnp.int32, xt.shape, 1)
    ctype = lane % 3
    far0 = jnp.abs(xt - _C0) < jnp.abs(xt - _C63)
    hi = far0.astype(jnp.float32)
    om = (jnp.abs(xt) > _HPI).astype(jnp.float32)
    f = jnp.where(ctype == 0, 4.0 * hi, jnp.where(ctype == 1, 2.0 * hi, om))
    cell = f + _shift_left(f, 1) + _shift_left(f, 2)

    m0 = ctype == 0
    for c in range(8):
        m = m0 & (cell == np.float32(c))
        acc_s[c:c + 1, :] += jnp.sum(jnp.where(m, r, 0.0), axis=0, keepdims=True)
        acc_n[c:c + 1, :] += jnp.sum(jnp.where(m, 1.0, 0.0), axis=0, keepdims=True)

    @pl.when(i == nblk - 1)
    def _():
        s = jnp.sum(acc_s[...], axis=1, keepdims=True)
        n = jnp.sum(acc_n[...], axis=1, keepdims=True)
        per_cell = jnp.where(n > 0, s / n, 0.0)
        out_ref[0, 0] = jnp.sum(per_cell) / np.float32(total_n)


def kernel(inputs, targets):
    n = inputs.shape[0]
    total = n * 3
    rows = total // _COLS
    nblk = rows // _BROWS
    xi = inputs.reshape(rows, _COLS)
    xt = targets.reshape(rows, _COLS)
    out = pl.pallas_call(
        functools.partial(_body, nblk, n),
        grid=(nblk,),
        in_specs=[
            pl.BlockSpec((_BROWS, _COLS), lambda i: (i, 0)),
            pl.BlockSpec((_BROWS, _COLS), lambda i: (i, 0)),
        ],
        out_specs=pl.BlockSpec((1, 1), lambda i: (0, 0), memory_space=pltpu.SMEM),
        out_shape=jax.ShapeDtypeStruct((1, 1), jnp.float32),
        scratch_shapes=[
            pltpu.VMEM((8, _COLS), jnp.float32),
            pltpu.VMEM((8, _COLS), jnp.float32),
        ],
    )(xi, xt)
    return out[0, 0]


# speedup vs baseline: 71.8628x; 34.6242x over previous
import functools

import numpy as np
import jax
import jax.numpy as jnp
from jax.experimental import pallas as pl
from jax.experimental.pallas import tpu as pltpu

_BINS = 64
_OFF = 2.0 * np.pi / _BINS
_C0 = np.float32(-np.pi + _OFF / 2.0)
_C63 = np.float32(-np.pi + _OFF / 2.0 + 63 * _OFF)
_HPI = np.float32(np.pi / 2.0)

_C = 51200


def _body(nblk, total_n, xi_ref, xt_ref, out_ref, acc_s, acc_n):
    i = pl.program_id(0)

    @pl.when(i == 0)
    def _():
        acc_s[...] = jnp.zeros_like(acc_s)
        acc_n[...] = jnp.zeros_like(acc_n)

    xi = xi_ref[...]
    xt = xt_ref[...]

    cat = jnp.concatenate([xi, xt], axis=0)
    s_all = jnp.sin(cat)
    c_all = jnp.cos(cat)
    ds = s_all[0:3, :] - s_all[3:6, :]
    dc = c_all[0:3, :] - c_all[3:6, :]
    nsin = jnp.sum(ds * ds, axis=0, keepdims=True)
    ncos = jnp.sum(dc * dc, axis=0, keepdims=True)
    r = jnp.sqrt(nsin) + jnp.sqrt(ncos)

    far0 = jnp.abs(xt - _C0) < jnp.abs(xt - _C63)
    om = jnp.abs(xt) > _HPI
    phi = jnp.where(far0[0:1, :], 4.0, 0.0)
    psi = jnp.where(far0[1:2, :], 2.0, 0.0)
    omg = jnp.where(om[2:3, :], 1.0, 0.0)
    cell = phi + psi + omg

    col = i * _C + jax.lax.broadcasted_iota(jnp.int32, (1, _C), 1)
    cell = jnp.where(col < total_n, cell, 8.0)

    rows = jax.lax.broadcasted_iota(jnp.int32, (8, _C), 0).astype(jnp.float32)
    oh = rows == cell
    acc_s[...] += jnp.where(oh, r, 0.0)
    acc_n[...] += jnp.where(oh, 1.0, 0.0)

    @pl.when(i == nblk - 1)
    def _():
        s = jnp.sum(acc_s[...], axis=1, keepdims=True)
        n = jnp.sum(acc_n[...], axis=1, keepdims=True)
        per_cell = jnp.where(n > 0, s / n, 0.0)
        out_ref[0, 0] = jnp.sum(per_cell) / np.float32(total_n)


def kernel(inputs, targets):
    n = inputs.shape[0]
    xi = inputs.T
    xt = targets.T
    nblk = (n + _C - 1) // _C
    out = pl.pallas_call(
        functools.partial(_body, nblk, n),
        grid=(nblk,),
        in_specs=[
            pl.BlockSpec((3, _C), lambda i: (0, i)),
            pl.BlockSpec((3, _C), lambda i: (0, i)),
        ],
        out_specs=pl.BlockSpec((1, 1), lambda i: (0, 0), memory_space=pltpu.SMEM),
        out_shape=jax.ShapeDtypeStruct((1, 1), jnp.float32),
        scratch_shapes=[
            pltpu.VMEM((8, _C), jnp.float32),
            pltpu.VMEM((8, _C), jnp.float32),
        ],
    )(xi, xt)
    return out[0, 0]


# speedup vs baseline: 148.8721x; 2.0716x over previous
import functools

import numpy as np
import jax
import jax.numpy as jnp
from jax.experimental import pallas as pl
from jax.experimental.pallas import tpu as pltpu

_BINS = 64
_OFF = 2.0 * np.pi / _BINS
_C0 = np.float32(-np.pi + _OFF / 2.0)
_C63 = np.float32(-np.pi + _OFF / 2.0 + 63 * _OFF)
_HPI = np.float32(np.pi / 2.0)

_C = 51200

_INV_2PI = np.float32(1.0 / (2.0 * np.pi))
_2PI = np.float32(2.0 * np.pi)
_COS_POLY = (
    np.float32(1.0), np.float32(-0.49999988079071045),
    np.float32(0.041666485369205475), np.float32(-0.0013887776294723153),
    np.float32(2.4769300580373965e-05), np.float32(-2.7073272690358863e-07),
    np.float32(1.7223942272437398e-09),
)


def _fast_cos(x):
    y = x - jnp.round(x * _INV_2PI) * _2PI
    z = y * y
    r = jnp.full_like(z, _COS_POLY[-1])
    for c in _COS_POLY[-2::-1]:
        r = r * z + c
    return r


def _body(nblk, total_n, xi_ref, xt_ref, out_ref, acc_s, acc_n):
    i = pl.program_id(0)

    @pl.when(i == 0)
    def _():
        acc_s[...] = jnp.zeros_like(acc_s)
        acc_n[...] = jnp.zeros_like(acc_n)

    xi = xi_ref[...]
    xt = xt_ref[...]

    cpq = _fast_cos(jnp.concatenate([xi + xt, xi - xt], axis=0))
    cp = cpq[0:3, :]
    cq = cpq[3:6, :]
    u = 1.0 - cq
    t = cp * u
    nsin = jnp.sum(u + t, axis=0, keepdims=True)
    ncos = jnp.sum(u - t, axis=0, keepdims=True)
    r = jnp.sqrt(nsin) + jnp.sqrt(ncos)

    far0 = jnp.abs(xt - _C0) < jnp.abs(xt - _C63)
    om = jnp.abs(xt) > _HPI
    phi = jnp.where(far0[0:1, :], 4.0, 0.0)
    psi = jnp.where(far0[1:2, :], 2.0, 0.0)
    omg = jnp.where(om[2:3, :], 1.0, 0.0)
    cell = phi + psi + omg

    col = i * _C + jax.lax.broadcasted_iota(jnp.int32, (1, _C), 1)
    cell = jnp.where(col < total_n, cell, 8.0)

    rows = jax.lax.broadcasted_iota(jnp.int32, (8, _C), 0).astype(jnp.float32)
    oh = rows == cell
    acc_s[...] += jnp.where(oh, r, 0.0)
    acc_n[...] += jnp.where(oh, 1.0, 0.0)

    @pl.when(i == nblk - 1)
    def _():
        s = jnp.sum(acc_s[...], axis=1, keepdims=True)
        n = jnp.sum(acc_n[...], axis=1, keepdims=True)
        per_cell = jnp.where(n > 0, s / n, 0.0)
        out_ref[0, 0] = jnp.sum(per_cell) / np.float32(total_n)


def kernel(inputs, targets):
    n = inputs.shape[0]
    xi = inputs.T
    xt = targets.T
    nblk = (n + _C - 1) // _C
    out = pl.pallas_call(
        functools.partial(_body, nblk, n),
        grid=(nblk,),
        in_specs=[
            pl.BlockSpec((3, _C), lambda i: (0, i)),
            pl.BlockSpec((3, _C), lambda i: (0, i)),
        ],
        out_specs=pl.BlockSpec((1, 1), lambda i: (0, 0), memory_space=pltpu.SMEM),
        out_shape=jax.ShapeDtypeStruct((1, 1), jnp.float32),
        scratch_shapes=[
            pltpu.VMEM((8, _C), jnp.float32),
            pltpu.VMEM((8, _C), jnp.float32),
        ],
    )(xi, xt)
    return out[0, 0]


# speedup vs baseline: 193.2899x; 1.2984x over previous
import functools

import numpy as np
import jax
import jax.numpy as jnp
from jax.experimental import pallas as pl
from jax.experimental.pallas import tpu as pltpu

_BINS = 64
_OFF = 2.0 * np.pi / _BINS
_C0 = np.float32(-np.pi + _OFF / 2.0)
_C63 = np.float32(-np.pi + _OFF / 2.0 + 63 * _OFF)
_HPI = np.float32(np.pi / 2.0)

_C = 51200

_INV_2PI = np.float32(1.0 / (2.0 * np.pi))
_2PI = np.float32(2.0 * np.pi)
_COS_POLY = (
    np.float32(1.0), np.float32(-0.49999988079071045),
    np.float32(0.041666485369205475), np.float32(-0.0013887776294723153),
    np.float32(2.4769300580373965e-05), np.float32(-2.7073272690358863e-07),
    np.float32(1.7223942272437398e-09),
)


def _fast_cos(x):
    y = x - jnp.round(x * _INV_2PI) * _2PI
    z = y * y
    r = jnp.full_like(z, _COS_POLY[-1])
    for c in _COS_POLY[-2::-1]:
        r = r * z + c
    return r


def _body(nblk, total_n, xi_ref, xt_ref, out_ref, acc_s, acc_n):
    i = pl.program_id(0)

    @pl.when(i == 0)
    def _():
        acc_s[...] = jnp.zeros_like(acc_s)
        acc_n[...] = jnp.zeros_like(acc_n)

    xi = xi_ref[...]
    xt = xt_ref[...]

    cpq = _fast_cos(jnp.concatenate([xi + xt, xi - xt], axis=0))
    cp = cpq[0:3, :]
    cq = cpq[3:6, :]
    u = 1.0 - cq
    t = cp * u

    ones13 = jnp.full((1, 3), 1.0, jnp.float32)
    nsin = jax.lax.dot(ones13, u + t, preferred_element_type=jnp.float32)
    ncos = jax.lax.dot(ones13, u - t, preferred_element_type=jnp.float32)
    nsin = nsin + np.float32(1e-30)
    ncos = ncos + np.float32(1e-30)
    r = nsin * jax.lax.rsqrt(nsin) + ncos * jax.lax.rsqrt(ncos)

    far0 = jnp.abs(xt - _C0) < jnp.abs(xt - _C63)
    om = jnp.abs(xt) > _HPI
    srow = jax.lax.broadcasted_iota(jnp.int32, (3, _C), 0)
    wvec = jnp.where(srow == 0, 4.0, jnp.where(srow == 1, 2.0, 1.0))
    is2 = srow == 2
    cond = (is2 & om) | (jnp.logical_not(is2) & far0)
    cellrows = jnp.where(cond, wvec, 0.0)
    cell = jax.lax.dot(ones13, cellrows, preferred_element_type=jnp.float32)

    col = i * _C + jax.lax.broadcasted_iota(jnp.int32, (1, _C), 1)
    cell = jnp.where(col < total_n, cell, 8.0)

    rows = jax.lax.broadcasted_iota(jnp.int32, (8, _C), 0).astype(jnp.float32)
    oh = rows == cell
    acc_s[...] += jnp.where(oh, r, 0.0)
    acc_n[...] += jnp.where(oh, 1.0, 0.0)

    @pl.when(i == nblk - 1)
    def _():
        s = jnp.sum(acc_s[...], axis=1, keepdims=True)
        n = jnp.sum(acc_n[...], axis=1, keepdims=True)
        per_cell = jnp.where(n > 0, s / n, 0.0)
        out_ref[0, 0] = jnp.sum(per_cell) / np.float32(total_n)


def kernel(inputs, targets):
    n = inputs.shape[0]
    xi = inputs.T
    xt = targets.T
    nblk = (n + _C - 1) // _C
    out = pl.pallas_call(
        functools.partial(_body, nblk, n),
        grid=(nblk,),
        in_specs=[
            pl.BlockSpec((3, _C), lambda i: (0, i)),
            pl.BlockSpec((3, _C), lambda i: (0, i)),
        ],
        out_specs=pl.BlockSpec((1, 1), lambda i: (0, 0), memory_space=pltpu.SMEM),
        out_shape=jax.ShapeDtypeStruct((1, 1), jnp.float32),
        scratch_shapes=[
            pltpu.VMEM((8, _C), jnp.float32),
            pltpu.VMEM((8, _C), jnp.float32),
        ],
    )(xi, xt)
    return out[0, 0]


# speedup vs baseline: 325.0161x; 1.6815x over previous
import functools

import numpy as np
import jax
import jax.numpy as jnp
from jax.experimental import pallas as pl
from jax.experimental.pallas import tpu as pltpu

_BINS = 64
_OFF = 2.0 * np.pi / _BINS
_C0 = np.float32(-np.pi + _OFF / 2.0)
_C63 = np.float32(-np.pi + _OFF / 2.0 + 63 * _OFF)
_HPI = np.float32(np.pi / 2.0)

_C = 51200
_C8 = _C // 8

_INV_2PI = np.float32(1.0 / (2.0 * np.pi))
_2PI = np.float32(2.0 * np.pi)
_COS_POLY = (
    np.float32(1.0), np.float32(-0.49999988079071045),
    np.float32(0.041666485369205475), np.float32(-0.0013887776294723153),
    np.float32(2.4769300580373965e-05), np.float32(-2.7073272690358863e-07),
    np.float32(1.7223942272437398e-09),
)


def _fast_cos(x):
    y = x - jnp.round(x * _INV_2PI) * _2PI
    z = y * y
    r = jnp.full_like(z, _COS_POLY[-1])
    for c in _COS_POLY[-2::-1]:
        r = r * z + c
    return r


def _restack(a):
    return jnp.concatenate([a[:, j * _C8:(j + 1) * _C8] for j in range(8)],
                           axis=0)


def _body(nblk, total_n, xi_ref, xt_ref, out_ref, acc_s, acc_n):
    i = pl.program_id(0)

    @pl.when(i == 0)
    def _():
        acc_s[...] = jnp.zeros_like(acc_s)
        acc_n[...] = jnp.zeros_like(acc_n)

    xis = _restack(xi_ref[...])
    xts = _restack(xt_ref[...])

    pcat = jnp.concatenate([xis + xts, xis - xts], axis=0)
    pcat = jnp.where(jnp.abs(pcat) < np.float32(1e4), pcat, 0.0)
    cpq = _fast_cos(pcat)
    cp = cpq[0:24, :]
    cq = cpq[24:48, :]
    u = 1.0 - cq
    t = cp * u

    m24 = jax.lax.broadcasted_iota(jnp.int32, (8, 24), 1)
    r8 = jax.lax.broadcasted_iota(jnp.int32, (8, 24), 0)
    sel = (m24 // 3 == r8).astype(jnp.float32)
    nsin = jax.lax.dot(sel, u + t, preferred_element_type=jnp.float32)
    ncos = jax.lax.dot(sel, u - t, preferred_element_type=jnp.float32)
    nsin = jnp.maximum(nsin, np.float32(1e-30))
    ncos = jnp.maximum(ncos, np.float32(1e-30))
    r = nsin * jax.lax.rsqrt(nsin) + ncos * jax.lax.rsqrt(ncos)

    far0 = jnp.abs(xts - _C0) < jnp.abs(xts - _C63)
    om = jnp.abs(xts) > _HPI
    srow = jax.lax.broadcasted_iota(jnp.int32, (24, _C8), 0)
    comp = srow - (srow // 3) * 3
    wvec = jnp.where(comp == 0, 4.0, jnp.where(comp == 1, 2.0, 1.0))
    is2 = comp == 2
    cond = (is2 & om) | (jnp.logical_not(is2) & far0)
    cellrows = jnp.where(cond, wvec, 0.0)
    cell = jax.lax.dot(sel, cellrows, preferred_element_type=jnp.float32)

    col = (i * _C
           + jax.lax.broadcasted_iota(jnp.int32, (8, _C8), 0) * _C8
           + jax.lax.broadcasted_iota(jnp.int32, (8, _C8), 1))
    cell = jnp.where(col < total_n, cell, 8.0)

    for c in range(8):
        m = cell == np.float32(c)
        acc_s[c] += jnp.where(m, r, 0.0)
        acc_n[c] += jnp.where(m, 1.0, 0.0)

    @pl.when(i == nblk - 1)
    def _():
        tot = jnp.float32(0.0)
        for c in range(8):
            s = jnp.sum(acc_s[c])
            n = jnp.sum(acc_n[c])
            tot += jnp.where(n > 0, s / jnp.maximum(n, 1.0), 0.0)
        out_ref[0, 0] = tot / np.float32(total_n)


def kernel(inputs, targets):
    n = inputs.shape[0]
    xi = inputs.T
    xt = targets.T
    nblk = (n + _C - 1) // _C
    out = pl.pallas_call(
        functools.partial(_body, nblk, n),
        grid=(nblk,),
        in_specs=[
            pl.BlockSpec((3, _C), lambda i: (0, i)),
            pl.BlockSpec((3, _C), lambda i: (0, i)),
        ],
        out_specs=pl.BlockSpec((1, 1), lambda i: (0, 0), memory_space=pltpu.SMEM),
        out_shape=jax.ShapeDtypeStruct((1, 1), jnp.float32),
        scratch_shapes=[
            pltpu.VMEM((8, 8, _C8), jnp.float32),
            pltpu.VMEM((8, 8, _C8), jnp.float32),
        ],
    )(xi, xt)
    return out[0, 0]
